# baseline (speedup 1.0000x reference)
import jax
import jax.numpy as jnp
from jax import lax
from jax.experimental import pallas as pl


def _malconv_kernel(zt_ref, w1_ref, w2_ref, b1_ref, b2_ref, out_ref):
    zbt = zt_ref[0]
    e, tc = zbt.shape
    nw = tc // 512
    zz = zbt.reshape(e, nw, 512).transpose(1, 0, 2).reshape(nw, 512 * e)
    dn = (((1,), (1,)), ((), ()))
    c1 = lax.dot_general(zz, w1_ref[...], dn, preferred_element_type=jnp.float32)
    c2 = lax.dot_general(zz, w2_ref[...], dn, preferred_element_type=jnp.float32)
    g = (c1 + b1_ref[...]) * jax.nn.sigmoid(c2 + b2_ref[...])
    out_ref[0] = jnp.max(g, axis=0, keepdims=True)


def kernel(z, W1, b1, W2, b2):
    B, T, E = z.shape
    C, _, K = W1.shape
    KE = K * E
    zt = jnp.transpose(z, (0, 2, 1))
    W1v = W1.reshape(C, KE)
    W2v = W2.reshape(C, KE)
    out = pl.pallas_call(
        _malconv_kernel,
        grid=(B,),
        in_specs=[
            pl.BlockSpec((1, E, T), lambda b: (b, 0, 0)),
            pl.BlockSpec((C, KE), lambda b: (0, 0)),
            pl.BlockSpec((C, KE), lambda b: (0, 0)),
            pl.BlockSpec((1, C), lambda b: (0, 0)),
            pl.BlockSpec((1, C), lambda b: (0, 0)),
        ],
        out_specs=pl.BlockSpec((1, 1, C), lambda b: (b, 0, 0)),
        out_shape=jax.ShapeDtypeStruct((B, 1, C), jnp.float32),
    )(zt, W1v, W2v, b1.reshape(1, C), b2.reshape(1, C))
    return out.reshape(B, C)
